# baseline (speedup 1.0000x reference)
import functools

import jax
import jax.numpy as jnp
from jax import lax
from jax.experimental import pallas as pl
from jax.experimental.pallas import tpu as pltpu
from jax.experimental.pallas import tpu_sc as plsc

_SIGMA = 1.0
_TW = 144
_CHUNK = 128
_NWORKERS = 32
_BPTS = 400
_KAPPA = 32


_NBUF = 4


def _sc_gather(table, idxs):
    etot = idxs.shape[0]
    epw = etot // _NWORKERS
    ngroups = epw // (_CHUNK * _NBUF)
    mesh = plsc.VectorSubcoreMesh(core_axis_name="c", subcore_axis_name="s")

    @functools.partial(
        pl.kernel,
        mesh=mesh,
        out_type=jax.ShapeDtypeStruct((etot, _TW), jnp.float32),
        scratch_types=[
            pltpu.VMEM((_NBUF, _CHUNK), jnp.int32),
            pltpu.VMEM((_NBUF, _CHUNK, _TW), jnp.float32),
            [pltpu.SemaphoreType.DMA] * _NBUF,
            [pltpu.SemaphoreType.DMA] * _NBUF,
        ],
        compiler_params=pltpu.CompilerParams(use_tc_tiling_on_sc=False),
    )
    def gather_kernel(tab_hbm, idx_hbm, out_hbm, idx_v, rows_v, gsem, wsem):
        wid = lax.axis_index("s") * 2 + lax.axis_index("c")
        base = wid * epw

        def fire(g, b):
            off = base + (g * _NBUF + b) * _CHUNK
            pltpu.sync_copy(idx_hbm.at[pl.ds(off, _CHUNK)], idx_v.at[b])
            pltpu.async_copy(tab_hbm.at[idx_v.at[b]], rows_v.at[b], gsem[b])

        for b in range(_NBUF):
            fire(0, b)

        def body(g, carry):
            for b in range(_NBUF):
                off = base + (g * _NBUF + b) * _CHUNK
                pltpu.make_async_copy(tab_hbm.at[idx_v.at[b]],
                                      rows_v.at[b], gsem[b]).wait()
                pltpu.async_copy(rows_v.at[b],
                                 out_hbm.at[pl.ds(off, _CHUNK)], wsem[b])
            for b in range(_NBUF):
                pltpu.make_async_copy(
                    rows_v.at[b],
                    out_hbm.at[pl.ds(base + (g * _NBUF + b) * _CHUNK, _CHUNK)],
                    wsem[b]).wait()
                fire(g + 1, b)
            return carry

        lax.fori_loop(0, ngroups - 1, body, 0)

        for b in range(_NBUF):
            off = base + ((ngroups - 1) * _NBUF + b) * _CHUNK
            pltpu.make_async_copy(tab_hbm.at[idx_v.at[b]],
                                  rows_v.at[b], gsem[b]).wait()
            pltpu.sync_copy(rows_v.at[b], out_hbm.at[pl.ds(off, _CHUNK)])

    return gather_kernel(table, idxs)


def _tc_body(g_ref, x_ref, qt_ref, q2_ref, wf_ref, y_ref):
    b = y_ref.shape[0]
    kappa = g_ref.shape[0] // b
    din = wf_ref.shape[0] // qt_ref.shape[1]
    nx = qt_ref.shape[0]
    mq = qt_ref.shape[1]
    e = b * kappa

    g = g_ref[...]
    fn = g[:, :din]
    xn = g[:, din:din + nx]
    xc = jnp.repeat(x_ref[...], kappa, axis=0)
    diff = xn - xc
    dd = jnp.sum(diff * diff, axis=1, keepdims=True)
    dq = jnp.dot(diff, qt_ref[...],
                 precision=lax.Precision.HIGHEST,
                 preferred_element_type=jnp.float32)
    d2 = jnp.maximum(dd - 2.0 * dq + q2_ref[...], 0.0)
    dist = jnp.sqrt(d2 + 1e-12)
    infl = jnp.maximum(0.0, 1.0 - dist / _SIGMA)

    parts = []
    for m in range(mq):
        w = infl[:, m:m + 1]
        t = (w * fn).reshape(b, kappa, din)
        parts.append(jnp.sum(t, axis=1))
    agg = jnp.concatenate(parts, axis=1)
    y_ref[...] = jnp.dot(agg, wf_ref[...],
                         precision=lax.Precision.HIGHEST,
                         preferred_element_type=jnp.float32)


def _tc_compute(g, x2, qt, q2, wf, r, dout):
    nblocks = r // _BPTS
    eblk = _BPTS * _KAPPA

    return pl.pallas_call(
        _tc_body,
        grid=(nblocks,),
        in_specs=[
            pl.BlockSpec((eblk, _TW), lambda i: (i, 0)),
            pl.BlockSpec((_BPTS, x2.shape[1]), lambda i: (i, 0)),
            pl.BlockSpec(qt.shape, lambda i: (0, 0)),
            pl.BlockSpec(q2.shape, lambda i: (0, 0)),
            pl.BlockSpec(wf.shape, lambda i: (0, 0)),
        ],
        out_specs=pl.BlockSpec((_BPTS, dout), lambda i: (i, 0)),
        out_shape=jax.ShapeDtypeStruct((r, dout), jnp.float32),
        compiler_params=pltpu.CompilerParams(
            dimension_semantics=("arbitrary",),
        ),
    )(g, x2, qt, q2, wf)


_NSLICE = 5


def kernel(X, F, N, Q, W):
    k, r, nx = X.shape
    kappa = N.shape[2]
    mq, din, dout = W.shape
    x2 = X[0]
    f2 = F[0]

    pad = jnp.zeros((r, _TW - din - nx), jnp.float32)
    table = jnp.concatenate([f2, x2, pad], axis=1)

    nflat = N[0].reshape(-1)

    qt = Q.T
    q2 = jnp.sum(Q * Q, axis=1)[None, :]
    wf = W.reshape(mq * din, dout)

    rs = r // _NSLICE
    es = rs * kappa
    egrain = _NWORKERS * _CHUNK * _NBUF
    etot = ((es + egrain - 1) // egrain) * egrain
    zpad = jnp.zeros((etot - es,), jnp.int32)

    ys = []
    for s in range(_NSLICE):
        idx_s = lax.dynamic_slice_in_dim(nflat, s * es, es)
        idx_s = jnp.concatenate([idx_s, zpad])
        g_s = _sc_gather(table, idx_s)
        x_s = lax.dynamic_slice_in_dim(x2, s * rs, rs)
        ys.append(_tc_compute(g_s, x_s, qt, q2, wf, rs, dout))
    y = jnp.concatenate(ys, axis=0)
    return y.reshape(k, r, dout)


# speedup vs baseline: 1.3666x; 1.3666x over previous
import functools

import jax
import jax.numpy as jnp
from jax import lax
from jax.experimental import pallas as pl
from jax.experimental.pallas import tpu as pltpu
from jax.experimental.pallas import tpu_sc as plsc

_SIGMA = 1.0
_TW = 144
_CHUNK = 128
_NWORKERS = 32
_BPTS = 200
_KAPPA = 32
_GRP = 8


_NBUF = 4


def _sc_gather(table, idxs):
    etot = idxs.shape[0]
    epw = etot // _NWORKERS
    ngroups = epw // (_CHUNK * _NBUF)
    mesh = plsc.VectorSubcoreMesh(core_axis_name="c", subcore_axis_name="s")

    @functools.partial(
        pl.kernel,
        mesh=mesh,
        out_type=jax.ShapeDtypeStruct((etot, _TW), jnp.float32),
        scratch_types=[
            pltpu.VMEM((_NBUF, _CHUNK), jnp.int32),
            pltpu.VMEM((_NBUF, _CHUNK, _TW), jnp.float32),
            [pltpu.SemaphoreType.DMA] * _NBUF,
            [pltpu.SemaphoreType.DMA] * _NBUF,
        ],
        compiler_params=pltpu.CompilerParams(use_tc_tiling_on_sc=False),
    )
    def gather_kernel(tab_hbm, idx_hbm, out_hbm, idx_v, rows_v, gsem, wsem):
        wid = lax.axis_index("s") * 2 + lax.axis_index("c")
        base = wid * epw

        def fire(g, b):
            off = base + (g * _NBUF + b) * _CHUNK
            pltpu.sync_copy(idx_hbm.at[pl.ds(off, _CHUNK)], idx_v.at[b])
            pltpu.async_copy(tab_hbm.at[idx_v.at[b]], rows_v.at[b], gsem[b])

        for b in range(_NBUF):
            fire(0, b)

        def body(g, carry):
            for b in range(_NBUF):
                off = base + (g * _NBUF + b) * _CHUNK
                pltpu.make_async_copy(tab_hbm.at[idx_v.at[b]],
                                      rows_v.at[b], gsem[b]).wait()
                pltpu.async_copy(rows_v.at[b],
                                 out_hbm.at[pl.ds(off, _CHUNK)], wsem[b])
            for b in range(_NBUF):
                pltpu.make_async_copy(
                    rows_v.at[b],
                    out_hbm.at[pl.ds(base + (g * _NBUF + b) * _CHUNK, _CHUNK)],
                    wsem[b]).wait()
                fire(g + 1, b)
            return carry

        lax.fori_loop(0, ngroups - 1, body, 0)

        for b in range(_NBUF):
            off = base + ((ngroups - 1) * _NBUF + b) * _CHUNK
            pltpu.make_async_copy(tab_hbm.at[idx_v.at[b]],
                                  rows_v.at[b], gsem[b]).wait()
            pltpu.sync_copy(rows_v.at[b], out_hbm.at[pl.ds(off, _CHUNK)])

    return gather_kernel(table, idxs)


def _tc_body(g_ref, x_ref, qt_ref, q2_ref, wf_ref, y_ref):
    b = y_ref.shape[0]
    kappa = g_ref.shape[0] // b
    din = wf_ref.shape[0] // qt_ref.shape[1]
    nx = qt_ref.shape[0]
    mq = qt_ref.shape[1]
    e = b * kappa

    g = g_ref[...]
    fn = g[:, :din]
    xn = g[:, din:din + nx]
    xc = jnp.repeat(x_ref[...], kappa, axis=0)
    diff = xn - xc
    dd = jnp.sum(diff * diff, axis=1, keepdims=True)
    dq = jnp.dot(diff, qt_ref[...],
                 precision=lax.Precision.HIGHEST,
                 preferred_element_type=jnp.float32)
    d2 = jnp.maximum(dd - 2.0 * dq + q2_ref[...], 0.0)
    dist = jnp.sqrt(d2 + 1e-12)
    infl = jnp.maximum(0.0, 1.0 - dist / _SIGMA)

    gp = _GRP
    ngr = b // gp
    cw = gp * mq
    rows = lax.broadcasted_iota(jnp.int32, (e, cw), 0)
    cols = lax.broadcasted_iota(jnp.int32, (e, cw), 1)
    mask = ((rows // kappa) % gp) == (cols // mq)
    tiled = jnp.concatenate([infl] * gp, axis=1)
    v = jnp.where(mask, tiled, 0.0)
    parts = []
    for gi in range(ngr):
        lo, hi = gi * gp * kappa, (gi + 1) * gp * kappa
        parts.append(lax.dot_general(
            v[lo:hi, :], fn[lo:hi, :],
            (((0,), (0,)), ((), ())),
            precision=lax.Precision.HIGHEST,
            preferred_element_type=jnp.float32))
    agg = jnp.concatenate(parts, axis=0).reshape(b, mq * din)
    y_ref[...] = jnp.dot(agg, wf_ref[...],
                         precision=lax.Precision.HIGHEST,
                         preferred_element_type=jnp.float32)


def _tc_compute(g, x2, qt, q2, wf, r, dout):
    nblocks = r // _BPTS
    eblk = _BPTS * _KAPPA

    return pl.pallas_call(
        _tc_body,
        grid=(nblocks,),
        in_specs=[
            pl.BlockSpec((eblk, _TW), lambda i: (i, 0)),
            pl.BlockSpec((_BPTS, x2.shape[1]), lambda i: (i, 0)),
            pl.BlockSpec(qt.shape, lambda i: (0, 0)),
            pl.BlockSpec(q2.shape, lambda i: (0, 0)),
            pl.BlockSpec(wf.shape, lambda i: (0, 0)),
        ],
        out_specs=pl.BlockSpec((_BPTS, dout), lambda i: (i, 0)),
        out_shape=jax.ShapeDtypeStruct((r, dout), jnp.float32),
        compiler_params=pltpu.CompilerParams(
            dimension_semantics=("arbitrary",),
        ),
    )(g, x2, qt, q2, wf)


_NSLICE = 1


def kernel(X, F, N, Q, W):
    k, r, nx = X.shape
    kappa = N.shape[2]
    mq, din, dout = W.shape
    x2 = X[0]
    f2 = F[0]

    pad = jnp.zeros((r, _TW - din - nx), jnp.float32)
    table = jnp.concatenate([f2, x2, pad], axis=1)

    nflat = N[0].reshape(-1)

    qt = Q.T
    q2 = jnp.sum(Q * Q, axis=1)[None, :]
    wf = W.reshape(mq * din, dout)

    rs = r // _NSLICE
    es = rs * kappa
    egrain = _NWORKERS * _CHUNK * _NBUF
    etot = ((es + egrain - 1) // egrain) * egrain
    zpad = jnp.zeros((etot - es,), jnp.int32)

    ys = []
    for s in range(_NSLICE):
        idx_s = lax.dynamic_slice_in_dim(nflat, s * es, es)
        idx_s = jnp.concatenate([idx_s, zpad])
        g_s = _sc_gather(table, idx_s)
        x_s = lax.dynamic_slice_in_dim(x2, s * rs, rs)
        ys.append(_tc_compute(g_s, x_s, qt, q2, wf, rs, dout))
    y = jnp.concatenate(ys, axis=0)
    return y.reshape(k, r, dout)


# speedup vs baseline: 1.5284x; 1.1184x over previous
import functools

import jax
import jax.numpy as jnp
from jax import lax
from jax.experimental import pallas as pl
from jax.experimental.pallas import tpu as pltpu
from jax.experimental.pallas import tpu_sc as plsc

_SIGMA = 1.0
_TW = 144
_CHUNK = 128
_NWORKERS = 32
_BPTS = 200
_KAPPA = 32
_GRP = 8


_NBUF = 4


def _sc_gather(table, idxs):
    etot = idxs.shape[0]
    epw = etot // _NWORKERS
    ngroups = epw // (_CHUNK * _NBUF)
    mesh = plsc.VectorSubcoreMesh(core_axis_name="c", subcore_axis_name="s")

    @functools.partial(
        pl.kernel,
        mesh=mesh,
        out_type=jax.ShapeDtypeStruct((etot, _TW), jnp.float32),
        scratch_types=[
            pltpu.VMEM((_NBUF, _CHUNK), jnp.int32),
            pltpu.VMEM((_NBUF, _CHUNK, _TW), jnp.float32),
            [pltpu.SemaphoreType.DMA] * _NBUF,
            [pltpu.SemaphoreType.DMA] * _NBUF,
        ],
        compiler_params=pltpu.CompilerParams(use_tc_tiling_on_sc=False),
    )
    def gather_kernel(tab_hbm, idx_hbm, out_hbm, idx_v, rows_v, gsem, wsem):
        wid = lax.axis_index("s") * 2 + lax.axis_index("c")
        base = wid * epw

        def fire(g, b):
            off = base + (g * _NBUF + b) * _CHUNK
            pltpu.sync_copy(idx_hbm.at[pl.ds(off, _CHUNK)], idx_v.at[b])
            pltpu.async_copy(tab_hbm.at[idx_v.at[b]], rows_v.at[b], gsem[b])

        for b in range(_NBUF):
            fire(0, b)

        def body(g, carry):
            for b in range(_NBUF):
                off = base + (g * _NBUF + b) * _CHUNK
                pltpu.make_async_copy(tab_hbm.at[idx_v.at[b]],
                                      rows_v.at[b], gsem[b]).wait()
                pltpu.async_copy(rows_v.at[b],
                                 out_hbm.at[pl.ds(off, _CHUNK)], wsem[b])
            for b in range(_NBUF):
                pltpu.make_async_copy(
                    rows_v.at[b],
                    out_hbm.at[pl.ds(base + (g * _NBUF + b) * _CHUNK, _CHUNK)],
                    wsem[b]).wait()
                fire(g + 1, b)
            return carry

        lax.fori_loop(0, ngroups - 1, body, 0)

        for b in range(_NBUF):
            off = base + ((ngroups - 1) * _NBUF + b) * _CHUNK
            pltpu.make_async_copy(tab_hbm.at[idx_v.at[b]],
                                  rows_v.at[b], gsem[b]).wait()
            pltpu.sync_copy(rows_v.at[b], out_hbm.at[pl.ds(off, _CHUNK)])

    return gather_kernel(table, idxs)


def _tc_body(g_ref, x_ref, qt_ref, q2_ref, wf_ref, y_ref):
    b = y_ref.shape[0]
    kappa = g_ref.shape[0] // b
    din = wf_ref.shape[0] // qt_ref.shape[1]
    nx = qt_ref.shape[0]
    mq = qt_ref.shape[1]
    e = b * kappa

    g = g_ref[...]
    fn = g[:, :din]
    xn = g[:, din:din + nx]
    xc = jnp.repeat(x_ref[...], kappa, axis=0)
    diff = xn - xc
    dd = jnp.sum(diff * diff, axis=1, keepdims=True)
    dq = jnp.dot(diff, qt_ref[...],
                 precision=lax.Precision.HIGHEST,
                 preferred_element_type=jnp.float32)
    d2 = jnp.maximum(dd - 2.0 * dq + q2_ref[...], 0.0)
    dist = jnp.sqrt(d2 + 1e-12)
    infl = jnp.maximum(0.0, 1.0 - dist / _SIGMA)

    gp = _GRP
    ngr = b // gp
    cw = gp * mq
    rows = lax.broadcasted_iota(jnp.int32, (e, cw), 0)
    cols = lax.broadcasted_iota(jnp.int32, (e, cw), 1)
    mask = ((rows // kappa) % gp) == (cols // mq)
    tiled = jnp.concatenate([infl] * gp, axis=1)
    v = jnp.where(mask, tiled, 0.0)
    parts = []
    for gi in range(ngr):
        lo, hi = gi * gp * kappa, (gi + 1) * gp * kappa
        parts.append(lax.dot_general(
            v[lo:hi, :], fn[lo:hi, :],
            (((0,), (0,)), ((), ())),
            preferred_element_type=jnp.float32))
    agg = jnp.concatenate(parts, axis=0).reshape(b, mq * din)
    y_ref[...] = jnp.dot(agg, wf_ref[...],
                         precision=lax.Precision.HIGHEST,
                         preferred_element_type=jnp.float32)


def _tc_compute(g, x2, qt, q2, wf, r, dout):
    nblocks = r // _BPTS
    eblk = _BPTS * _KAPPA

    return pl.pallas_call(
        _tc_body,
        grid=(nblocks,),
        in_specs=[
            pl.BlockSpec((eblk, _TW), lambda i: (i, 0)),
            pl.BlockSpec((_BPTS, x2.shape[1]), lambda i: (i, 0)),
            pl.BlockSpec(qt.shape, lambda i: (0, 0)),
            pl.BlockSpec(q2.shape, lambda i: (0, 0)),
            pl.BlockSpec(wf.shape, lambda i: (0, 0)),
        ],
        out_specs=pl.BlockSpec((_BPTS, dout), lambda i: (i, 0)),
        out_shape=jax.ShapeDtypeStruct((r, dout), jnp.float32),
        compiler_params=pltpu.CompilerParams(
            dimension_semantics=("arbitrary",),
        ),
    )(g, x2, qt, q2, wf)


_NSLICE = 1


def kernel(X, F, N, Q, W):
    k, r, nx = X.shape
    kappa = N.shape[2]
    mq, din, dout = W.shape
    x2 = X[0]
    f2 = F[0]

    pad = jnp.zeros((r, _TW - din - nx), jnp.float32)
    table = jnp.concatenate([f2, x2, pad], axis=1)

    nflat = N[0].reshape(-1)

    qt = Q.T
    q2 = jnp.sum(Q * Q, axis=1)[None, :]
    wf = W.reshape(mq * din, dout)

    rs = r // _NSLICE
    es = rs * kappa
    egrain = _NWORKERS * _CHUNK * _NBUF
    etot = ((es + egrain - 1) // egrain) * egrain
    zpad = jnp.zeros((etot - es,), jnp.int32)

    ys = []
    for s in range(_NSLICE):
        idx_s = lax.dynamic_slice_in_dim(nflat, s * es, es)
        idx_s = jnp.concatenate([idx_s, zpad])
        g_s = _sc_gather(table, idx_s)
        x_s = lax.dynamic_slice_in_dim(x2, s * rs, rs)
        ys.append(_tc_compute(g_s, x_s, qt, q2, wf, rs, dout))
    y = jnp.concatenate(ys, axis=0)
    return y.reshape(k, r, dout)


# speedup vs baseline: 1.7729x; 1.1599x over previous
import functools

import jax
import jax.numpy as jnp
from jax import lax
from jax.experimental import pallas as pl
from jax.experimental.pallas import tpu as pltpu
from jax.experimental.pallas import tpu_sc as plsc

_SIGMA = 1.0
_TW = 144
_CHUNK = 128
_NWORKERS = 32
_BPTS = 400
_KAPPA = 32
_GRP = 8


_NBUF = 4


def _sc_gather(table, idxs):
    etot = idxs.shape[0]
    epw = etot // _NWORKERS
    ngroups = epw // (_CHUNK * _NBUF)
    mesh = plsc.VectorSubcoreMesh(core_axis_name="c", subcore_axis_name="s")

    @functools.partial(
        pl.kernel,
        mesh=mesh,
        out_type=jax.ShapeDtypeStruct((etot, _TW), jnp.float32),
        scratch_types=[
            pltpu.VMEM((_NBUF, _CHUNK), jnp.int32),
            pltpu.VMEM((_NBUF, _CHUNK, _TW), jnp.float32),
            [pltpu.SemaphoreType.DMA] * _NBUF,
            [pltpu.SemaphoreType.DMA] * _NBUF,
        ],
        compiler_params=pltpu.CompilerParams(use_tc_tiling_on_sc=False),
    )
    def gather_kernel(tab_hbm, idx_hbm, out_hbm, idx_v, rows_v, gsem, wsem):
        wid = lax.axis_index("s") * 2 + lax.axis_index("c")
        base = wid * epw

        def fire(g, b):
            off = base + (g * _NBUF + b) * _CHUNK
            pltpu.sync_copy(idx_hbm.at[pl.ds(off, _CHUNK)], idx_v.at[b])
            pltpu.async_copy(tab_hbm.at[idx_v.at[b]], rows_v.at[b], gsem[b])

        for b in range(_NBUF):
            fire(0, b)

        def body(g, carry):
            for b in range(_NBUF):
                off = base + (g * _NBUF + b) * _CHUNK
                pltpu.make_async_copy(tab_hbm.at[idx_v.at[b]],
                                      rows_v.at[b], gsem[b]).wait()
                pltpu.async_copy(rows_v.at[b],
                                 out_hbm.at[pl.ds(off, _CHUNK)], wsem[b])
            for b in range(_NBUF):
                pltpu.make_async_copy(
                    rows_v.at[b],
                    out_hbm.at[pl.ds(base + (g * _NBUF + b) * _CHUNK, _CHUNK)],
                    wsem[b]).wait()
                fire(g + 1, b)
            return carry

        lax.fori_loop(0, ngroups - 1, body, 0)

        for b in range(_NBUF):
            off = base + ((ngroups - 1) * _NBUF + b) * _CHUNK
            pltpu.make_async_copy(tab_hbm.at[idx_v.at[b]],
                                  rows_v.at[b], gsem[b]).wait()
            pltpu.sync_copy(rows_v.at[b], out_hbm.at[pl.ds(off, _CHUNK)])

    return gather_kernel(table, idxs)


def _tc_body(g_ref, x_ref, qt_ref, q2_ref, wf_ref, y_ref):
    b = y_ref.shape[0]
    kappa = g_ref.shape[0] // b
    din = wf_ref.shape[0] // qt_ref.shape[1]
    nx = qt_ref.shape[0]
    mq = qt_ref.shape[1]
    e = b * kappa

    g = g_ref[...]
    fn = g[:, :din]
    xn = g[:, din:din + nx]
    xc = jnp.repeat(x_ref[...], kappa, axis=0)
    diff = xn - xc
    dd = jnp.sum(diff * diff, axis=1, keepdims=True)
    dq = jnp.dot(diff, qt_ref[...],
                 precision=lax.Precision.HIGHEST,
                 preferred_element_type=jnp.float32)
    d2 = jnp.maximum(dd - 2.0 * dq + q2_ref[...], 0.0)
    dist = jnp.sqrt(d2 + 1e-12)
    infl = jnp.maximum(0.0, 1.0 - dist / _SIGMA)

    gp = _GRP
    ngr = b // gp
    cw = gp * mq
    rows = lax.broadcasted_iota(jnp.int32, (e, cw), 0)
    cols = lax.broadcasted_iota(jnp.int32, (e, cw), 1)
    mask = ((rows // kappa) % gp) == (cols // mq)
    tiled = jnp.concatenate([infl] * gp, axis=1)
    v = jnp.where(mask, tiled, 0.0)
    parts = []
    for gi in range(ngr):
        lo, hi = gi * gp * kappa, (gi + 1) * gp * kappa
        parts.append(lax.dot_general(
            v[lo:hi, :], fn[lo:hi, :],
            (((0,), (0,)), ((), ())),
            preferred_element_type=jnp.float32))
    agg = jnp.concatenate(parts, axis=0).reshape(b, mq * din)
    y_ref[...] = jnp.dot(agg, wf_ref[...],
                         preferred_element_type=jnp.float32)


def _tc_compute(g, x2, qt, q2, wf, r, dout):
    nblocks = r // _BPTS
    eblk = _BPTS * _KAPPA

    return pl.pallas_call(
        _tc_body,
        grid=(nblocks,),
        in_specs=[
            pl.BlockSpec((eblk, _TW), lambda i: (i, 0)),
            pl.BlockSpec((_BPTS, x2.shape[1]), lambda i: (i, 0)),
            pl.BlockSpec(qt.shape, lambda i: (0, 0)),
            pl.BlockSpec(q2.shape, lambda i: (0, 0)),
            pl.BlockSpec(wf.shape, lambda i: (0, 0)),
        ],
        out_specs=pl.BlockSpec((_BPTS, dout), lambda i: (i, 0)),
        out_shape=jax.ShapeDtypeStruct((r, dout), jnp.float32),
        compiler_params=pltpu.CompilerParams(
            dimension_semantics=("arbitrary",),
            vmem_limit_bytes=110 * 1024 * 1024,
        ),
    )(g, x2, qt, q2, wf)


_NSLICE = 1


def kernel(X, F, N, Q, W):
    k, r, nx = X.shape
    kappa = N.shape[2]
    mq, din, dout = W.shape
    x2 = X[0]
    f2 = F[0]

    pad = jnp.zeros((r, _TW - din - nx), jnp.float32)
    table = jnp.concatenate([f2, x2, pad], axis=1)

    nflat = N[0].reshape(-1)

    qt = Q.T
    q2 = jnp.sum(Q * Q, axis=1)[None, :]
    wf = W.reshape(mq * din, dout)

    rs = r // _NSLICE
    es = rs * kappa
    egrain = _NWORKERS * _CHUNK * _NBUF
    etot = ((es + egrain - 1) // egrain) * egrain
    zpad = jnp.zeros((etot - es,), jnp.int32)

    ys = []
    for s in range(_NSLICE):
        idx_s = lax.dynamic_slice_in_dim(nflat, s * es, es)
        idx_s = jnp.concatenate([idx_s, zpad])
        g_s = _sc_gather(table, idx_s)
        x_s = lax.dynamic_slice_in_dim(x2, s * rs, rs)
        ys.append(_tc_compute(g_s, x_s, qt, q2, wf, rs, dout))
    y = jnp.concatenate(ys, axis=0)
    return y.reshape(k, r, dout)


# speedup vs baseline: 1.8310x; 1.0328x over previous
import functools

import jax
import jax.numpy as jnp
from jax import lax
from jax.experimental import pallas as pl
from jax.experimental.pallas import tpu as pltpu
from jax.experimental.pallas import tpu_sc as plsc

_SIGMA = 1.0
_XW = 16
_CHUNK = 128
_NWORKERS = 32
_BPTS = 400
_KAPPA = 32
_GRP = 8


_NBUF = 4


def _sc_gather(tabf, tabx, idxs):
    etot = idxs.shape[0]
    din = tabf.shape[1]
    epw = etot // _NWORKERS
    ngroups = epw // (_CHUNK * _NBUF)
    mesh = plsc.VectorSubcoreMesh(core_axis_name="c", subcore_axis_name="s")

    @functools.partial(
        pl.kernel,
        mesh=mesh,
        out_type=(jax.ShapeDtypeStruct((etot, din), jnp.bfloat16),
                  jax.ShapeDtypeStruct((etot, _XW), jnp.float32)),
        scratch_types=[
            pltpu.VMEM((_NBUF, _CHUNK), jnp.int32),
            pltpu.VMEM((_NBUF, _CHUNK, din), jnp.bfloat16),
            pltpu.VMEM((_NBUF, _CHUNK, _XW), jnp.float32),
            [pltpu.SemaphoreType.DMA] * _NBUF,
            [pltpu.SemaphoreType.DMA] * _NBUF,
            [pltpu.SemaphoreType.DMA] * _NBUF,
            [pltpu.SemaphoreType.DMA] * _NBUF,
        ],
        compiler_params=pltpu.CompilerParams(use_tc_tiling_on_sc=False),
    )
    def gather_kernel(tf_hbm, tx_hbm, idx_hbm, of_hbm, ox_hbm,
                      idx_v, rf_v, rx_v, gfsem, gxsem, wfsem, wxsem):
        wid = lax.axis_index("s") * 2 + lax.axis_index("c")
        base = wid * epw

        def fire(g, b):
            off = base + (g * _NBUF + b) * _CHUNK
            pltpu.sync_copy(idx_hbm.at[pl.ds(off, _CHUNK)], idx_v.at[b])
            pltpu.async_copy(tf_hbm.at[idx_v.at[b]], rf_v.at[b], gfsem[b])
            pltpu.async_copy(tx_hbm.at[idx_v.at[b]], rx_v.at[b], gxsem[b])

        for b in range(_NBUF):
            fire(0, b)

        def body(g, carry):
            for b in range(_NBUF):
                off = base + (g * _NBUF + b) * _CHUNK
                pltpu.make_async_copy(tf_hbm.at[idx_v.at[b]],
                                      rf_v.at[b], gfsem[b]).wait()
                pltpu.make_async_copy(tx_hbm.at[idx_v.at[b]],
                                      rx_v.at[b], gxsem[b]).wait()
                pltpu.async_copy(rf_v.at[b],
                                 of_hbm.at[pl.ds(off, _CHUNK)], wfsem[b])
                pltpu.async_copy(rx_v.at[b],
                                 ox_hbm.at[pl.ds(off, _CHUNK)], wxsem[b])
            for b in range(_NBUF):
                off = base + (g * _NBUF + b) * _CHUNK
                pltpu.make_async_copy(rf_v.at[b],
                                      of_hbm.at[pl.ds(off, _CHUNK)],
                                      wfsem[b]).wait()
                pltpu.make_async_copy(rx_v.at[b],
                                      ox_hbm.at[pl.ds(off, _CHUNK)],
                                      wxsem[b]).wait()
                fire(g + 1, b)
            return carry

        lax.fori_loop(0, ngroups - 1, body, 0)

        for b in range(_NBUF):
            off = base + ((ngroups - 1) * _NBUF + b) * _CHUNK
            pltpu.make_async_copy(tf_hbm.at[idx_v.at[b]],
                                  rf_v.at[b], gfsem[b]).wait()
            pltpu.make_async_copy(tx_hbm.at[idx_v.at[b]],
                                  rx_v.at[b], gxsem[b]).wait()
            pltpu.sync_copy(rf_v.at[b], of_hbm.at[pl.ds(off, _CHUNK)])
            pltpu.sync_copy(rx_v.at[b], ox_hbm.at[pl.ds(off, _CHUNK)])

    return gather_kernel(tabf, tabx, idxs)


def _tc_body(gf_ref, gx_ref, x_ref, qt_ref, q2_ref, wf_ref, y_ref):
    b = y_ref.shape[0]
    kappa = gf_ref.shape[0] // b
    din = wf_ref.shape[0] // qt_ref.shape[1]
    nx = qt_ref.shape[0]
    mq = qt_ref.shape[1]
    e = b * kappa

    fn = gf_ref[...]
    xn = gx_ref[...][:, :nx]
    xc = jnp.repeat(x_ref[...], kappa, axis=0)
    diff = xn - xc
    dd = jnp.sum(diff * diff, axis=1, keepdims=True)
    dq = jnp.dot(diff, qt_ref[...],
                 precision=lax.Precision.HIGHEST,
                 preferred_element_type=jnp.float32)
    d2 = jnp.maximum(dd - 2.0 * dq + q2_ref[...], 0.0)
    dist = jnp.sqrt(d2 + 1e-12)
    infl = jnp.maximum(0.0, 1.0 - dist / _SIGMA)

    gp = _GRP
    ngr = b // gp
    cw = gp * mq
    rows = lax.broadcasted_iota(jnp.int32, (e, cw), 0)
    cols = lax.broadcasted_iota(jnp.int32, (e, cw), 1)
    mask = ((rows // kappa) % gp) == (cols // mq)
    inflb = infl.astype(jnp.bfloat16)
    tiled = jnp.concatenate([inflb] * gp, axis=1)
    v = jnp.where(mask, tiled, jnp.bfloat16(0.0))
    parts = []
    for gi in range(ngr):
        lo, hi = gi * gp * kappa, (gi + 1) * gp * kappa
        parts.append(lax.dot_general(
            v[lo:hi, :], fn[lo:hi, :],
            (((0,), (0,)), ((), ())),
            preferred_element_type=jnp.float32))
    agg = jnp.concatenate(parts, axis=0).reshape(b, mq * din)
    y_ref[...] = jnp.dot(agg, wf_ref[...],
                         preferred_element_type=jnp.float32)


def _tc_compute(gf, gx, x2, qt, q2, wf, r, dout):
    nblocks = r // _BPTS
    eblk = _BPTS * _KAPPA

    return pl.pallas_call(
        _tc_body,
        grid=(nblocks,),
        in_specs=[
            pl.BlockSpec((eblk, gf.shape[1]), lambda i: (i, 0)),
            pl.BlockSpec((eblk, _XW), lambda i: (i, 0)),
            pl.BlockSpec((_BPTS, x2.shape[1]), lambda i: (i, 0)),
            pl.BlockSpec(qt.shape, lambda i: (0, 0)),
            pl.BlockSpec(q2.shape, lambda i: (0, 0)),
            pl.BlockSpec(wf.shape, lambda i: (0, 0)),
        ],
        out_specs=pl.BlockSpec((_BPTS, dout), lambda i: (i, 0)),
        out_shape=jax.ShapeDtypeStruct((r, dout), jnp.float32),
        compiler_params=pltpu.CompilerParams(
            dimension_semantics=("arbitrary",),
            vmem_limit_bytes=110 * 1024 * 1024,
        ),
    )(gf, gx, x2, qt, q2, wf)


_NSLICE = 1


def kernel(X, F, N, Q, W):
    k, r, nx = X.shape
    kappa = N.shape[2]
    mq, din, dout = W.shape
    x2 = X[0]
    f2 = F[0]

    tabf = f2.astype(jnp.bfloat16)
    tabx = jnp.concatenate(
        [x2, jnp.zeros((r, _XW - nx), jnp.float32)], axis=1)

    nflat = N[0].reshape(-1)

    qt = Q.T
    q2 = jnp.sum(Q * Q, axis=1)[None, :]
    wf = W.reshape(mq * din, dout)

    rs = r // _NSLICE
    es = rs * kappa
    egrain = _NWORKERS * _CHUNK * _NBUF
    etot = ((es + egrain - 1) // egrain) * egrain
    zpad = jnp.zeros((etot - es,), jnp.int32)

    ys = []
    for s in range(_NSLICE):
        idx_s = lax.dynamic_slice_in_dim(nflat, s * es, es)
        idx_s = jnp.concatenate([idx_s, zpad])
        gf_s, gx_s = _sc_gather(tabf, tabx, idx_s)
        x_s = lax.dynamic_slice_in_dim(x2, s * rs, rs)
        ys.append(_tc_compute(gf_s, gx_s, x_s, qt, q2, wf, rs, dout))
    y = jnp.concatenate(ys, axis=0)
    return y.reshape(k, r, dout)


# speedup vs baseline: 1.8518x; 1.0113x over previous
import functools

import jax
import jax.numpy as jnp
from jax import lax
from jax.experimental import pallas as pl
from jax.experimental.pallas import tpu as pltpu
from jax.experimental.pallas import tpu_sc as plsc

_SIGMA = 1.0
_XW = 16
_CHUNK = 256
_NWORKERS = 32
_BPTS = 400
_KAPPA = 32
_GRP = 8


_NBUF = 4


def _sc_gather(tabf, tabx, idxs):
    etot = idxs.shape[0]
    din = tabf.shape[1]
    epw = etot // _NWORKERS
    ngroups = epw // (_CHUNK * _NBUF)
    mesh = plsc.VectorSubcoreMesh(core_axis_name="c", subcore_axis_name="s")

    @functools.partial(
        pl.kernel,
        mesh=mesh,
        out_type=(jax.ShapeDtypeStruct((etot, din), jnp.bfloat16),
                  jax.ShapeDtypeStruct((etot, _XW), jnp.float32)),
        scratch_types=[
            pltpu.VMEM((_NBUF, _CHUNK), jnp.int32),
            pltpu.VMEM((_NBUF, _CHUNK, din), jnp.bfloat16),
            pltpu.VMEM((_NBUF, _CHUNK, _XW), jnp.float32),
            [pltpu.SemaphoreType.DMA] * _NBUF,
            [pltpu.SemaphoreType.DMA] * _NBUF,
            [pltpu.SemaphoreType.DMA] * _NBUF,
            [pltpu.SemaphoreType.DMA] * _NBUF,
        ],
        compiler_params=pltpu.CompilerParams(use_tc_tiling_on_sc=False),
    )
    def gather_kernel(tf_hbm, tx_hbm, idx_hbm, of_hbm, ox_hbm,
                      idx_v, rf_v, rx_v, gfsem, gxsem, wfsem, wxsem):
        wid = lax.axis_index("s") * 2 + lax.axis_index("c")
        base = wid * epw

        def fire(g, b):
            off = base + (g * _NBUF + b) * _CHUNK
            pltpu.sync_copy(idx_hbm.at[pl.ds(off, _CHUNK)], idx_v.at[b])
            pltpu.async_copy(tf_hbm.at[idx_v.at[b]], rf_v.at[b], gfsem[b])
            pltpu.async_copy(tx_hbm.at[idx_v.at[b]], rx_v.at[b], gxsem[b])

        for b in range(_NBUF):
            fire(0, b)

        def body(g, carry):
            for b in range(_NBUF):
                off = base + (g * _NBUF + b) * _CHUNK
                pltpu.make_async_copy(tf_hbm.at[idx_v.at[b]],
                                      rf_v.at[b], gfsem[b]).wait()
                pltpu.make_async_copy(tx_hbm.at[idx_v.at[b]],
                                      rx_v.at[b], gxsem[b]).wait()
                pltpu.async_copy(rf_v.at[b],
                                 of_hbm.at[pl.ds(off, _CHUNK)], wfsem[b])
                pltpu.async_copy(rx_v.at[b],
                                 ox_hbm.at[pl.ds(off, _CHUNK)], wxsem[b])
            for b in range(_NBUF):
                off = base + (g * _NBUF + b) * _CHUNK
                pltpu.make_async_copy(rf_v.at[b],
                                      of_hbm.at[pl.ds(off, _CHUNK)],
                                      wfsem[b]).wait()
                pltpu.make_async_copy(rx_v.at[b],
                                      ox_hbm.at[pl.ds(off, _CHUNK)],
                                      wxsem[b]).wait()
                fire(g + 1, b)
            return carry

        lax.fori_loop(0, ngroups - 1, body, 0)

        for b in range(_NBUF):
            off = base + ((ngroups - 1) * _NBUF + b) * _CHUNK
            pltpu.make_async_copy(tf_hbm.at[idx_v.at[b]],
                                  rf_v.at[b], gfsem[b]).wait()
            pltpu.make_async_copy(tx_hbm.at[idx_v.at[b]],
                                  rx_v.at[b], gxsem[b]).wait()
            pltpu.sync_copy(rf_v.at[b], of_hbm.at[pl.ds(off, _CHUNK)])
            pltpu.sync_copy(rx_v.at[b], ox_hbm.at[pl.ds(off, _CHUNK)])

    return gather_kernel(tabf, tabx, idxs)


def _tc_body(gf_ref, gx_ref, x_ref, qt_ref, q2_ref, wf_ref, y_ref):
    b = y_ref.shape[0]
    kappa = gf_ref.shape[0] // b
    din = wf_ref.shape[0] // qt_ref.shape[1]
    nx = qt_ref.shape[0]
    mq = qt_ref.shape[1]
    e = b * kappa

    fn = gf_ref[...]
    xn = gx_ref[...][:, :nx]
    xc = jnp.repeat(x_ref[...], kappa, axis=0)
    diff = xn - xc
    dd = jnp.sum(diff * diff, axis=1, keepdims=True)
    dq = jnp.dot(diff, qt_ref[...],
                 precision=lax.Precision.HIGHEST,
                 preferred_element_type=jnp.float32)
    d2 = jnp.maximum(dd - 2.0 * dq + q2_ref[...], 0.0)
    dist = jnp.sqrt(d2 + 1e-12)
    infl = jnp.maximum(0.0, 1.0 - dist / _SIGMA)

    gp = _GRP
    ngr = b // gp
    cw = gp * mq
    rows = lax.broadcasted_iota(jnp.int32, (e, cw), 0)
    cols = lax.broadcasted_iota(jnp.int32, (e, cw), 1)
    mask = ((rows // kappa) % gp) == (cols // mq)
    inflb = infl.astype(jnp.bfloat16)
    tiled = jnp.concatenate([inflb] * gp, axis=1)
    v = jnp.where(mask, tiled, jnp.bfloat16(0.0))
    parts = []
    for gi in range(ngr):
        lo, hi = gi * gp * kappa, (gi + 1) * gp * kappa
        parts.append(lax.dot_general(
            v[lo:hi, :], fn[lo:hi, :],
            (((0,), (0,)), ((), ())),
            preferred_element_type=jnp.float32))
    agg = jnp.concatenate(parts, axis=0).reshape(b, mq * din)
    y_ref[...] = jnp.dot(agg, wf_ref[...],
                         preferred_element_type=jnp.float32)


def _tc_compute(gf, gx, x2, qt, q2, wf, r, dout):
    nblocks = r // _BPTS
    eblk = _BPTS * _KAPPA

    return pl.pallas_call(
        _tc_body,
        grid=(nblocks,),
        in_specs=[
            pl.BlockSpec((eblk, gf.shape[1]), lambda i: (i, 0)),
            pl.BlockSpec((eblk, _XW), lambda i: (i, 0)),
            pl.BlockSpec((_BPTS, x2.shape[1]), lambda i: (i, 0)),
            pl.BlockSpec(qt.shape, lambda i: (0, 0)),
            pl.BlockSpec(q2.shape, lambda i: (0, 0)),
            pl.BlockSpec(wf.shape, lambda i: (0, 0)),
        ],
        out_specs=pl.BlockSpec((_BPTS, dout), lambda i: (i, 0)),
        out_shape=jax.ShapeDtypeStruct((r, dout), jnp.float32),
        compiler_params=pltpu.CompilerParams(
            dimension_semantics=("arbitrary",),
            vmem_limit_bytes=110 * 1024 * 1024,
        ),
    )(gf, gx, x2, qt, q2, wf)


_NSLICE = 1


def kernel(X, F, N, Q, W):
    k, r, nx = X.shape
    kappa = N.shape[2]
    mq, din, dout = W.shape
    x2 = X[0]
    f2 = F[0]

    tabf = f2.astype(jnp.bfloat16)
    tabx = jnp.concatenate(
        [x2, jnp.zeros((r, _XW - nx), jnp.float32)], axis=1)

    nflat = N[0].reshape(-1)

    qt = Q.T
    q2 = jnp.sum(Q * Q, axis=1)[None, :]
    wf = W.reshape(mq * din, dout)

    rs = r // _NSLICE
    es = rs * kappa
    egrain = _NWORKERS * _CHUNK * _NBUF
    etot = ((es + egrain - 1) // egrain) * egrain
    zpad = jnp.zeros((etot - es,), jnp.int32)

    ys = []
    for s in range(_NSLICE):
        idx_s = lax.dynamic_slice_in_dim(nflat, s * es, es)
        idx_s = jnp.concatenate([idx_s, zpad])
        gf_s, gx_s = _sc_gather(tabf, tabx, idx_s)
        x_s = lax.dynamic_slice_in_dim(x2, s * rs, rs)
        ys.append(_tc_compute(gf_s, gx_s, x_s, qt, q2, wf, rs, dout))
    y = jnp.concatenate(ys, axis=0)
    return y.reshape(k, r, dout)
